# Initial kernel scaffold; baseline (speedup 1.0000x reference)
import jax
import jax.numpy as jnp
from jax.experimental import pallas as pl


def kernel(x, edge_index, W1, b1, W2, b2):
    raise NotImplementedError("write your pallas kernel here")


# speedup vs baseline: 20.4371x; 20.4371x over previous
import functools

import jax
import jax.numpy as jnp
from jax import lax
from jax.experimental import pallas as pl
from jax.experimental.pallas import tpu as pltpu
from jax.experimental.pallas import tpu_sc as plsc

N = 10000
NP = 10240
D = 128
DH = D // 2
E = 320000
NC = 2
NS = 16
EPT_D = E // (NC * NS)
EPT_S = E // NS
K = 80
NCH_D = EPT_D // K
NCH_S = EPT_S // K
RPT = NP // NS

_mesh = plsc.VectorSubcoreMesh(core_axis_name="c", subcore_axis_name="s")


@functools.partial(
    pl.kernel,
    out_type=jax.ShapeDtypeStruct((NC, NP, 16), jnp.float32),
    mesh=_mesh,
    compiler_params=pltpu.CompilerParams(use_tc_tiling_on_sc=False),
    scratch_types=[
        pltpu.VMEM((NCH_D, K), jnp.int32),
        pltpu.VMEM((K, 16), jnp.float32),
        pltpu.VMEM((128, 16), jnp.float32),
        pltpu.VMEM_SHARED((NP, 16), jnp.float32),
    ],
)
def _deg_kernel(dst_hbm, out_hbm, dstb, onesb, zbuf, acc):
    cid = lax.axis_index("c")
    sid = lax.axis_index("s")
    wid = sid * NC + cid

    one16 = jnp.ones((16,), jnp.float32)
    zero16 = jnp.zeros((16,), jnp.float32)

    def fill_ones(i, _):
        onesb[i] = one16
        return 0

    lax.fori_loop(0, K, fill_ones, 0)

    def fill_zero(i, _):
        zbuf[i] = zero16
        return 0

    lax.fori_loop(0, 128, fill_zero, 0)
    for b in range(RPT // 128):
        pltpu.sync_copy(zbuf, acc.at[pl.ds(sid * RPT + b * 128, 128)])
    plsc.subcore_barrier()

    pltpu.sync_copy(dst_hbm.at[wid], dstb)

    def body(j, _):
        pltpu.sync_copy(onesb, acc.at[dstb.at[j]], add=True)
        return 0

    lax.fori_loop(0, NCH_D, body, 0)
    plsc.subcore_barrier()
    pltpu.sync_copy(acc.at[pl.ds(sid * RPT, RPT)],
                    out_hbm.at[cid, pl.ds(sid * RPT, RPT)])


@functools.partial(
    pl.kernel,
    out_type=jax.ShapeDtypeStruct((NC, NP, DH), jnp.float32),
    mesh=_mesh,
    compiler_params=pltpu.CompilerParams(use_tc_tiling_on_sc=False),
    scratch_types=[
        pltpu.VMEM((NCH_S, K), jnp.int32),
        pltpu.VMEM((NCH_S, K), jnp.int32),
        pltpu.VMEM((2, K, DH), jnp.float32),
        pltpu.VMEM((128, DH), jnp.float32),
        pltpu.VMEM_SHARED((NP, DH), jnp.float32),
        pltpu.SemaphoreType.DMA,
        pltpu.SemaphoreType.DMA,
    ],
)
def _scatter_kernel(h_hbm, src_hbm, dst_hbm, out_hbm, srcb, dstb, rows, zbuf,
                    acc, gsem, ssem):
    cid = lax.axis_index("c")
    sid = lax.axis_index("s")

    zero16 = jnp.zeros((16,), jnp.float32)

    def fill_zero(i, _):
        def col(j, _):
            zbuf[i, pl.ds(j * 16, 16)] = zero16
            return 0

        return lax.fori_loop(0, DH // 16, col, 0)

    lax.fori_loop(0, 128, fill_zero, 0)
    for b in range(RPT // 128):
        pltpu.sync_copy(zbuf, acc.at[pl.ds(sid * RPT + b * 128, 128)])
    plsc.subcore_barrier()

    pltpu.sync_copy(src_hbm.at[sid], srcb)
    pltpu.sync_copy(dst_hbm.at[sid], dstb)

    def gstart(j, buf):
        pltpu.async_copy(h_hbm.at[cid].at[srcb.at[j]], rows.at[buf], gsem)

    def gwait(buf):
        pltpu.make_async_copy(h_hbm.at[cid].at[srcb.at[0]], rows.at[buf],
                              gsem).wait()

    def sstart(j, buf):
        pltpu.async_copy(rows.at[buf], acc.at[dstb.at[j]], ssem, add=True)

    def swait(buf):
        pltpu.make_async_copy(rows.at[buf], acc.at[dstb.at[0]], ssem).wait()

    gstart(0, 0)
    gstart(1, 1)

    def pair(p, _):
        c0 = 2 * p
        gwait(0)
        sstart(c0, 0)
        gwait(1)
        swait(0)
        gstart(jnp.minimum(c0 + 2, NCH_S - 1), 0)
        sstart(c0 + 1, 1)
        swait(1)
        gstart(jnp.minimum(c0 + 3, NCH_S - 1), 1)
        return 0

    lax.fori_loop(0, NCH_S // 2, pair, 0)
    gwait(0)
    gwait(1)
    plsc.subcore_barrier()
    pltpu.sync_copy(acc.at[pl.ds(sid * RPT, RPT)],
                    out_hbm.at[cid, pl.ds(sid * RPT, RPT)])


_R = 1024


def _dis(d0_ref, d1_ref):
    deg = d0_ref[:, 0:1] + d1_ref[:, 0:1] + 1.0
    return lax.rsqrt(deg)


def _mm_scale_body(d0_ref, d1_ref, x_ref, w_ref, oa_ref, ob_ref):
    dis = _dis(d0_ref, d1_ref)
    h = dis * jnp.dot(x_ref[...], w_ref[...],
                      preferred_element_type=jnp.float32)
    oa_ref[...] = h[:, :DH]
    ob_ref[...] = h[:, DH:]


def _mid_body(d0_ref, d1_ref, sa_ref, sb_ref, ha_ref, hb_ref, b_ref, w_ref,
              oa_ref, ob_ref):
    dis = _dis(d0_ref, d1_ref)
    za = dis * (sa_ref[...] + ha_ref[...]) + b_ref[:, :DH]
    zb = dis * (sb_ref[...] + hb_ref[...]) + b_ref[:, DH:]
    a = jnp.concatenate([jnp.maximum(za, 0.0), jnp.maximum(zb, 0.0)], axis=1)
    h = dis * jnp.dot(a, w_ref[...], preferred_element_type=jnp.float32)
    oa_ref[...] = h[:, :DH]
    ob_ref[...] = h[:, DH:]


def _final_body(d0_ref, d1_ref, ta_ref, tb_ref, ha_ref, hb_ref, b_ref, o_ref):
    dis = _dis(d0_ref, d1_ref)
    o_ref[...] = dis * jnp.concatenate(
        [ta_ref[...] + ha_ref[...], tb_ref[...] + hb_ref[...]], axis=1
    ) + b_ref[...]


def _rowspec(cols):
    return pl.BlockSpec((_R, cols), lambda i: (i, 0))


def _fullspec(r, c):
    return pl.BlockSpec((r, c), lambda i: (0, 0))


_GRID = (NP // _R,)
_OUT_H = [jax.ShapeDtypeStruct((NP, DH), jnp.float32),
          jax.ShapeDtypeStruct((NP, DH), jnp.float32)]

_mm_scale = pl.pallas_call(
    _mm_scale_body,
    grid=_GRID,
    in_specs=[_rowspec(16), _rowspec(16), _rowspec(D), _fullspec(D, D)],
    out_specs=[_rowspec(DH), _rowspec(DH)],
    out_shape=_OUT_H,
)

_mid = pl.pallas_call(
    _mid_body,
    grid=_GRID,
    in_specs=[_rowspec(16), _rowspec(16), _rowspec(DH), _rowspec(DH),
              _rowspec(DH), _rowspec(DH), _fullspec(1, D), _fullspec(D, D)],
    out_specs=[_rowspec(DH), _rowspec(DH)],
    out_shape=_OUT_H,
)

_final = pl.pallas_call(
    _final_body,
    grid=_GRID,
    in_specs=[_rowspec(16), _rowspec(16), _rowspec(DH), _rowspec(DH),
              _rowspec(DH), _rowspec(DH), _fullspec(1, D)],
    out_specs=_rowspec(D),
    out_shape=jax.ShapeDtypeStruct((NP, D), jnp.float32),
)


def kernel(x, edge_index, W1, b1, W2, b2):
    src_d = edge_index[0].astype(jnp.int32).reshape(NC * NS, NCH_D, K)
    dst_d = edge_index[1].astype(jnp.int32).reshape(NC * NS, NCH_D, K)
    src_s = edge_index[0].astype(jnp.int32).reshape(NS, NCH_S, K)
    dst_s = edge_index[1].astype(jnp.int32).reshape(NS, NCH_S, K)
    xp = jnp.pad(x, ((0, NP - N), (0, 0)))
    b1r = b1.reshape(1, D)
    b2r = b2.reshape(1, D)

    degp = _deg_kernel(dst_d)
    d0, d1 = degp[0], degp[1]

    h1a, h1b = _mm_scale(d0, d1, xp, W1)
    h1 = jnp.stack([h1a, h1b])
    s = _scatter_kernel(h1, src_s, dst_s)
    h2a, h2b = _mid(d0, d1, s[0], s[1], h1a, h1b, b1r, W2)
    h2 = jnp.stack([h2a, h2b])
    t = _scatter_kernel(h2, src_s, dst_s)
    outp = _final(d0, d1, t[0], t[1], h2a, h2b, b2r)
    return outp[:N]


# speedup vs baseline: 21.3976x; 1.0470x over previous
import functools

import jax
import jax.numpy as jnp
from jax import lax
from jax.experimental import pallas as pl
from jax.experimental.pallas import tpu as pltpu
from jax.experimental.pallas import tpu_sc as plsc

N = 10000
NP = 10240
D = 128
DH = D // 2
E = 320000
NC = 2
NS = 16
EPT_D = E // (NC * NS)
EPT_S = E // NS
K = 80
NCH_D = EPT_D // K
NCH_S = EPT_S // K
RPT = NP // NS

_mesh = plsc.VectorSubcoreMesh(core_axis_name="c", subcore_axis_name="s")


@functools.partial(
    pl.kernel,
    out_type=jax.ShapeDtypeStruct((NC, NP, 16), jnp.float32),
    mesh=_mesh,
    compiler_params=pltpu.CompilerParams(use_tc_tiling_on_sc=False),
    scratch_types=[
        pltpu.VMEM((NCH_D, K), jnp.int32),
        pltpu.VMEM((K, 16), jnp.float32),
        pltpu.VMEM((128, 16), jnp.float32),
        pltpu.VMEM_SHARED((NP, 16), jnp.float32),
    ],
)
def _deg_kernel(dst_hbm, out_hbm, dstb, onesb, zbuf, acc):
    cid = lax.axis_index("c")
    sid = lax.axis_index("s")
    wid = sid * NC + cid

    one16 = jnp.ones((16,), jnp.float32)
    zero16 = jnp.zeros((16,), jnp.float32)

    def fill_ones(i, _):
        onesb[i] = one16
        return 0

    lax.fori_loop(0, K, fill_ones, 0)

    def fill_zero(i, _):
        zbuf[i] = zero16
        return 0

    lax.fori_loop(0, 128, fill_zero, 0)
    for b in range(RPT // 128):
        pltpu.sync_copy(zbuf, acc.at[pl.ds(sid * RPT + b * 128, 128)])
    plsc.subcore_barrier()

    pltpu.sync_copy(dst_hbm.at[wid], dstb)

    def body(j, _):
        pltpu.sync_copy(onesb, acc.at[dstb.at[j]], add=True)
        return 0

    lax.fori_loop(0, NCH_D, body, 0)
    plsc.subcore_barrier()
    pltpu.sync_copy(acc.at[pl.ds(sid * RPT, RPT)],
                    out_hbm.at[cid, pl.ds(sid * RPT, RPT)])


@functools.partial(
    pl.kernel,
    out_type=jax.ShapeDtypeStruct((NC, NP, DH), jnp.float32),
    mesh=_mesh,
    compiler_params=pltpu.CompilerParams(use_tc_tiling_on_sc=False),
    scratch_types=[
        pltpu.VMEM((NCH_S, K), jnp.int32),
        pltpu.VMEM((NCH_S, K), jnp.int32),
        pltpu.VMEM((2, K, DH), jnp.float32),
        pltpu.VMEM((128, DH), jnp.float32),
        pltpu.VMEM_SHARED((NP, DH), jnp.float32),
        pltpu.SemaphoreType.DMA,
        pltpu.SemaphoreType.DMA,
    ],
)
def _scatter_kernel(h_hbm, src_hbm, dst_hbm, out_hbm, srcb, dstb, rows, zbuf,
                    acc, gsem, ssem):
    cid = lax.axis_index("c")
    sid = lax.axis_index("s")

    zero16 = jnp.zeros((16,), jnp.float32)

    def fill_zero(i, _):
        def col(j, _):
            zbuf[i, pl.ds(j * 16, 16)] = zero16
            return 0

        return lax.fori_loop(0, DH // 16, col, 0)

    lax.fori_loop(0, 128, fill_zero, 0)
    for b in range(RPT // 128):
        pltpu.sync_copy(zbuf, acc.at[pl.ds(sid * RPT + b * 128, 128)])
    plsc.subcore_barrier()

    pltpu.sync_copy(src_hbm.at[sid], srcb)
    pltpu.sync_copy(dst_hbm.at[sid], dstb)

    def gstart(j, buf):
        pltpu.async_copy(h_hbm.at[cid].at[srcb.at[j]], rows.at[buf], gsem)

    def gwait(buf):
        pltpu.make_async_copy(h_hbm.at[cid].at[srcb.at[0]], rows.at[buf],
                              gsem).wait()

    def sstart(j, buf):
        pltpu.async_copy(rows.at[buf], acc.at[dstb.at[j]], ssem, add=True)

    def swait(buf):
        pltpu.make_async_copy(rows.at[buf], acc.at[dstb.at[0]], ssem).wait()

    gstart(0, 0)
    gstart(1, 1)

    def pair(p, _):
        c0 = 2 * p
        gwait(0)
        sstart(c0, 0)
        gwait(1)
        swait(0)
        gstart(jnp.minimum(c0 + 2, NCH_S - 1), 0)
        sstart(c0 + 1, 1)
        swait(1)
        gstart(jnp.minimum(c0 + 3, NCH_S - 1), 1)
        return 0

    lax.fori_loop(0, NCH_S // 2, pair, 0)
    gwait(0)
    gwait(1)
    plsc.subcore_barrier()
    pltpu.sync_copy(acc.at[pl.ds(sid * RPT, RPT)],
                    out_hbm.at[cid, pl.ds(sid * RPT, RPT)])


_R = 1024
_RF = 1000


def _dis3(d_refs):
    d0_ref, d1_ref = d_refs
    deg = d0_ref[0, :, 0:1] + d1_ref[0, :, 0:1] + 1.0
    return lax.rsqrt(deg)


def _mm_scale_body(d0_ref, d1_ref, x_ref, w_ref, o_ref):
    dis = _dis3((d0_ref, d1_ref))
    o_ref[...] = (dis * jnp.dot(x_ref[...], w_ref[0],
                                preferred_element_type=jnp.float32))[None]


def _mid_body(d0_ref, d1_ref, sa_ref, sb_ref, ha_ref, hb_ref, b_ref, w_ref,
              o_ref):
    dis = _dis3((d0_ref, d1_ref))
    za = dis * (sa_ref[0] + ha_ref[0]) + b_ref[:, :DH]
    zb = dis * (sb_ref[0] + hb_ref[0]) + b_ref[:, DH:]
    a = jnp.concatenate([jnp.maximum(za, 0.0), jnp.maximum(zb, 0.0)], axis=1)
    o_ref[...] = (dis * jnp.dot(a, w_ref[0],
                                preferred_element_type=jnp.float32))[None]


def _final_body(d0_ref, d1_ref, ta_ref, tb_ref, ha_ref, hb_ref, b_ref, o_ref):
    dis = _dis3((d0_ref, d1_ref))
    o_ref[...] = dis * jnp.concatenate(
        [ta_ref[0] + ha_ref[0], tb_ref[0] + hb_ref[0]], axis=1
    ) + b_ref[...]


def _dspec(r):
    return [pl.BlockSpec((1, r, 16), lambda i, *_: (0, i, 0)),
            pl.BlockSpec((1, r, 16), lambda i, *_: (1, i, 0))]


def _hspecs(r):
    return [pl.BlockSpec((1, r, DH), lambda i, *_: (0, i, 0)),
            pl.BlockSpec((1, r, DH), lambda i, *_: (1, i, 0))]


_H_OUT = jax.ShapeDtypeStruct((NC, NP, DH), jnp.float32)

_mm_scale = pl.pallas_call(
    _mm_scale_body,
    grid=(NP // _R, NC),
    in_specs=_dspec(_R) + [
        pl.BlockSpec((_R, D), lambda i, c: (i, 0)),
        pl.BlockSpec((1, D, DH), lambda i, c: (c, 0, 0)),
    ],
    out_specs=pl.BlockSpec((1, _R, DH), lambda i, c: (c, i, 0)),
    out_shape=_H_OUT,
)

_mid = pl.pallas_call(
    _mid_body,
    grid=(NP // _R, NC),
    in_specs=_dspec(_R) + _hspecs(_R) + _hspecs(_R) + [
        pl.BlockSpec((1, D), lambda i, c: (0, 0)),
        pl.BlockSpec((1, D, DH), lambda i, c: (c, 0, 0)),
    ],
    out_specs=pl.BlockSpec((1, _R, DH), lambda i, c: (c, i, 0)),
    out_shape=_H_OUT,
)

_final = pl.pallas_call(
    _final_body,
    grid=(N // _RF,),
    in_specs=_dspec(_RF) + _hspecs(_RF) + _hspecs(_RF) + [
        pl.BlockSpec((1, D), lambda i: (0, 0)),
    ],
    out_specs=pl.BlockSpec((_RF, D), lambda i: (i, 0)),
    out_shape=jax.ShapeDtypeStruct((N, D), jnp.float32),
)


def kernel(x, edge_index, W1, b1, W2, b2):
    src_d = edge_index[0].astype(jnp.int32).reshape(NC * NS, NCH_D, K)
    dst_d = edge_index[1].astype(jnp.int32).reshape(NC * NS, NCH_D, K)
    src_s = edge_index[0].astype(jnp.int32).reshape(NS, NCH_S, K)
    dst_s = edge_index[1].astype(jnp.int32).reshape(NS, NCH_S, K)
    xp = jnp.pad(x, ((0, NP - N), (0, 0)))
    b1r = b1.reshape(1, D)
    b2r = b2.reshape(1, D)
    W1s = jnp.stack([W1[:, :DH], W1[:, DH:]])
    W2s = jnp.stack([W2[:, :DH], W2[:, DH:]])

    degp = _deg_kernel(dst_d)

    h1 = _mm_scale(degp, degp, xp, W1s)
    s = _scatter_kernel(h1, src_s, dst_s)
    h2 = _mid(degp, degp, s, s, h1, h1, b1r, W2s)
    t = _scatter_kernel(h2, src_s, dst_s)
    return _final(degp, degp, t, t, h2, h2, b2r)


# speedup vs baseline: 21.5295x; 1.0062x over previous
import functools

import jax
import jax.numpy as jnp
from jax import lax
from jax.experimental import pallas as pl
from jax.experimental.pallas import tpu as pltpu
from jax.experimental.pallas import tpu_sc as plsc

N = 10000
NP = 10240
D = 128
DH = D // 2
E = 320000
NC = 2
NS = 16
EPT_D = E // (NC * NS)
EPT_S = E // NS
K = 80
NCH_D = EPT_D // K
NCH_S = EPT_S // K
RPT = NP // NS

_mesh = plsc.VectorSubcoreMesh(core_axis_name="c", subcore_axis_name="s")


@functools.partial(
    pl.kernel,
    out_type=jax.ShapeDtypeStruct((NC, NP, 16), jnp.float32),
    mesh=_mesh,
    compiler_params=pltpu.CompilerParams(use_tc_tiling_on_sc=False),
    scratch_types=[
        pltpu.VMEM((NCH_D, K), jnp.int32),
        pltpu.VMEM((K, 16), jnp.float32),
        pltpu.VMEM((128, 16), jnp.float32),
        pltpu.VMEM_SHARED((NP, 16), jnp.float32),
    ],
)
def _deg_kernel(dst_hbm, out_hbm, dstb, onesb, zbuf, acc):
    cid = lax.axis_index("c")
    sid = lax.axis_index("s")
    wid = sid * NC + cid

    one16 = jnp.ones((16,), jnp.float32)
    zero16 = jnp.zeros((16,), jnp.float32)

    def fill_ones(i, _):
        onesb[i] = one16
        return 0

    lax.fori_loop(0, K, fill_ones, 0)

    def fill_zero(i, _):
        zbuf[i] = zero16
        return 0

    lax.fori_loop(0, 128, fill_zero, 0)
    for b in range(RPT // 128):
        pltpu.sync_copy(zbuf, acc.at[pl.ds(sid * RPT + b * 128, 128)])
    plsc.subcore_barrier()

    pltpu.sync_copy(dst_hbm.at[wid], dstb)

    def body(j, _):
        pltpu.sync_copy(onesb, acc.at[dstb.at[j]], add=True)
        return 0

    lax.fori_loop(0, NCH_D, body, 0)
    plsc.subcore_barrier()
    pltpu.sync_copy(acc.at[pl.ds(sid * RPT, RPT)],
                    out_hbm.at[cid, pl.ds(sid * RPT, RPT)])


@functools.partial(
    pl.kernel,
    out_type=jax.ShapeDtypeStruct((NC, NP, DH), jnp.float32),
    mesh=_mesh,
    compiler_params=pltpu.CompilerParams(use_tc_tiling_on_sc=False),
    scratch_types=[
        pltpu.VMEM((NCH_S, K), jnp.int32),
        pltpu.VMEM((NCH_S, K), jnp.int32),
        pltpu.VMEM((2, K, DH), jnp.float32),
        pltpu.VMEM_SHARED((NP, DH), jnp.float32),
        pltpu.SemaphoreType.DMA,
        pltpu.SemaphoreType.DMA,
    ],
)
def _scatter_kernel(h_hbm, src_hbm, dst_hbm, out_hbm, srcb, dstb, rows,
                    acc, gsem, ssem):
    cid = lax.axis_index("c")
    sid = lax.axis_index("s")

    pltpu.sync_copy(h_hbm.at[cid, pl.ds(sid * RPT, RPT)],
                    acc.at[pl.ds(sid * RPT, RPT)])
    plsc.subcore_barrier()

    pltpu.sync_copy(src_hbm.at[sid], srcb)
    pltpu.sync_copy(dst_hbm.at[sid], dstb)

    def gstart(j, buf):
        pltpu.async_copy(h_hbm.at[cid].at[srcb.at[j]], rows.at[buf], gsem)

    def gwait(buf):
        pltpu.make_async_copy(h_hbm.at[cid].at[srcb.at[0]], rows.at[buf],
                              gsem).wait()

    def sstart(j, buf):
        pltpu.async_copy(rows.at[buf], acc.at[dstb.at[j]], ssem, add=True)

    def swait(buf):
        pltpu.make_async_copy(rows.at[buf], acc.at[dstb.at[0]], ssem).wait()

    gstart(0, 0)
    gstart(1, 1)

    def pair(p, _):
        c0 = 2 * p
        gwait(0)
        sstart(c0, 0)
        gwait(1)
        swait(0)
        gstart(jnp.minimum(c0 + 2, NCH_S - 1), 0)
        sstart(c0 + 1, 1)
        swait(1)
        gstart(jnp.minimum(c0 + 3, NCH_S - 1), 1)
        return 0

    lax.fori_loop(0, NCH_S // 2, pair, 0)
    gwait(0)
    gwait(1)
    plsc.subcore_barrier()
    pltpu.sync_copy(acc.at[pl.ds(sid * RPT, RPT)],
                    out_hbm.at[cid, pl.ds(sid * RPT, RPT)])


_R = 1024
_RF = 1000


def _dis3(d_refs):
    d0_ref, d1_ref = d_refs
    deg = d0_ref[0, :, 0:1] + d1_ref[0, :, 0:1] + 1.0
    return lax.rsqrt(deg)


def _mm_scale_body(d0_ref, d1_ref, x_ref, w_ref, o_ref):
    dis = _dis3((d0_ref, d1_ref))
    o_ref[...] = (dis * jnp.dot(x_ref[...], w_ref[0],
                                preferred_element_type=jnp.float32))[None]


def _mid_body(d0_ref, d1_ref, sa_ref, sb_ref, b_ref, w_ref, o_ref):
    dis = _dis3((d0_ref, d1_ref))
    za = dis * sa_ref[0] + b_ref[:, :DH]
    zb = dis * sb_ref[0] + b_ref[:, DH:]
    a = jnp.concatenate([jnp.maximum(za, 0.0), jnp.maximum(zb, 0.0)], axis=1)
    o_ref[...] = (dis * jnp.dot(a, w_ref[0],
                                preferred_element_type=jnp.float32))[None]


def _final_body(d0_ref, d1_ref, ta_ref, tb_ref, b_ref, o_ref):
    dis = _dis3((d0_ref, d1_ref))
    o_ref[...] = dis * jnp.concatenate(
        [ta_ref[0], tb_ref[0]], axis=1
    ) + b_ref[...]


def _dspec(r):
    return [pl.BlockSpec((1, r, 16), lambda i, *_: (0, i, 0)),
            pl.BlockSpec((1, r, 16), lambda i, *_: (1, i, 0))]


def _hspecs(r):
    return [pl.BlockSpec((1, r, DH), lambda i, *_: (0, i, 0)),
            pl.BlockSpec((1, r, DH), lambda i, *_: (1, i, 0))]


_H_OUT = jax.ShapeDtypeStruct((NC, NP, DH), jnp.float32)

_mm_scale = pl.pallas_call(
    _mm_scale_body,
    grid=(NP // _R, NC),
    in_specs=_dspec(_R) + [
        pl.BlockSpec((_R, D), lambda i, c: (i, 0)),
        pl.BlockSpec((1, D, DH), lambda i, c: (c, 0, 0)),
    ],
    out_specs=pl.BlockSpec((1, _R, DH), lambda i, c: (c, i, 0)),
    out_shape=_H_OUT,
)

_mid = pl.pallas_call(
    _mid_body,
    grid=(NP // _R, NC),
    in_specs=_dspec(_R) + _hspecs(_R) + [
        pl.BlockSpec((1, D), lambda i, c: (0, 0)),
        pl.BlockSpec((1, D, DH), lambda i, c: (c, 0, 0)),
    ],
    out_specs=pl.BlockSpec((1, _R, DH), lambda i, c: (c, i, 0)),
    out_shape=_H_OUT,
)

_final = pl.pallas_call(
    _final_body,
    grid=(N // _RF,),
    in_specs=_dspec(_RF) + _hspecs(_RF) + [
        pl.BlockSpec((1, D), lambda i: (0, 0)),
    ],
    out_specs=pl.BlockSpec((_RF, D), lambda i: (i, 0)),
    out_shape=jax.ShapeDtypeStruct((N, D), jnp.float32),
)


def kernel(x, edge_index, W1, b1, W2, b2):
    src_d = edge_index[0].astype(jnp.int32).reshape(NC * NS, NCH_D, K)
    dst_d = edge_index[1].astype(jnp.int32).reshape(NC * NS, NCH_D, K)
    src_s = edge_index[0].astype(jnp.int32).reshape(NS, NCH_S, K)
    dst_s = edge_index[1].astype(jnp.int32).reshape(NS, NCH_S, K)
    xp = jnp.pad(x, ((0, NP - N), (0, 0)))
    b1r = b1.reshape(1, D)
    b2r = b2.reshape(1, D)
    W1s = jnp.stack([W1[:, :DH], W1[:, DH:]])
    W2s = jnp.stack([W2[:, :DH], W2[:, DH:]])

    degp = _deg_kernel(dst_d)

    h1 = _mm_scale(degp, degp, xp, W1s)
    s = _scatter_kernel(h1, src_s, dst_s)
    h2 = _mid(degp, degp, s, s, b1r, W2s)
    t = _scatter_kernel(h2, src_s, dst_s)
    return _final(degp, degp, t, t, b2r)


# speedup vs baseline: 26.0963x; 1.2121x over previous
import functools

import jax
import jax.numpy as jnp
from jax import lax
from jax.experimental import pallas as pl
from jax.experimental.pallas import tpu as pltpu
from jax.experimental.pallas import tpu_sc as plsc

N = 10000
NP = 10240
D = 128
DH = D // 2
E = 320000
NC = 2
NS = 16
EPT_D = E // (NC * NS)
EPT_S = E // NS
K = 80
NCH_D = EPT_D // K
KB = 160
NCH_S = EPT_S // KB
RPT = NP // NS

_mesh = plsc.VectorSubcoreMesh(core_axis_name="c", subcore_axis_name="s")


@functools.partial(
    pl.kernel,
    out_type=jax.ShapeDtypeStruct((NC, NP, 16), jnp.float32),
    mesh=_mesh,
    compiler_params=pltpu.CompilerParams(use_tc_tiling_on_sc=False),
    scratch_types=[
        pltpu.VMEM((NCH_D, K), jnp.int32),
        pltpu.VMEM((K, 16), jnp.float32),
        pltpu.VMEM((128, 16), jnp.float32),
        pltpu.VMEM_SHARED((NP, 16), jnp.float32),
    ],
)
def _deg_kernel(dst_hbm, out_hbm, dstb, onesb, zbuf, acc):
    cid = lax.axis_index("c")
    sid = lax.axis_index("s")
    wid = sid * NC + cid

    one16 = jnp.ones((16,), jnp.float32)
    zero16 = jnp.zeros((16,), jnp.float32)

    def fill_ones(i, _):
        onesb[i] = one16
        return 0

    lax.fori_loop(0, K, fill_ones, 0)

    def fill_zero(i, _):
        zbuf[i] = zero16
        return 0

    lax.fori_loop(0, 128, fill_zero, 0)
    for b in range(RPT // 128):
        pltpu.sync_copy(zbuf, acc.at[pl.ds(sid * RPT + b * 128, 128)])
    plsc.subcore_barrier()

    pltpu.sync_copy(dst_hbm.at[wid], dstb)

    def body(j, _):
        pltpu.sync_copy(onesb, acc.at[dstb.at[j]], add=True)
        return 0

    lax.fori_loop(0, NCH_D, body, 0)
    plsc.subcore_barrier()
    pltpu.sync_copy(acc.at[pl.ds(sid * RPT, RPT)],
                    out_hbm.at[cid, pl.ds(sid * RPT, RPT)])


@functools.partial(
    pl.kernel,
    out_type=jax.ShapeDtypeStruct((NC, NP, DH), jnp.float32),
    mesh=_mesh,
    compiler_params=pltpu.CompilerParams(use_tc_tiling_on_sc=False),
    scratch_types=[
        pltpu.VMEM((EPT_S,), jnp.int32),
        pltpu.VMEM((EPT_S,), jnp.int32),
        pltpu.VMEM((2, KB, DH), jnp.float32),
        pltpu.VMEM_SHARED((NP, DH), jnp.float32),
        pltpu.SemaphoreType.DMA,
        pltpu.SemaphoreType.DMA,
    ],
)
def _scatter_kernel(h_hbm, src_hbm, dst_hbm, out_hbm, srcb, dstb, rows,
                    acc, gsem, ssem):
    cid = lax.axis_index("c")
    sid = lax.axis_index("s")

    pltpu.sync_copy(h_hbm.at[cid, pl.ds(sid * RPT, RPT)],
                    acc.at[pl.ds(sid * RPT, RPT)])
    plsc.subcore_barrier()

    pltpu.sync_copy(src_hbm.at[sid], srcb)
    pltpu.sync_copy(dst_hbm.at[sid], dstb)

    def gstart(j, buf):
        pltpu.async_copy(h_hbm.at[cid].at[srcb.at[pl.ds(j * KB, KB)]],
                         rows.at[buf], gsem)

    def gwait(buf):
        pltpu.make_async_copy(h_hbm.at[cid].at[srcb.at[pl.ds(0, KB)]],
                              rows.at[buf], gsem).wait()

    def sstart(j, buf):
        pltpu.async_copy(rows.at[buf], acc.at[dstb.at[pl.ds(j * KB, KB)]],
                         ssem, add=True)

    def swait(buf):
        pltpu.make_async_copy(rows.at[buf], acc.at[dstb.at[pl.ds(0, KB)]],
                              ssem).wait()

    gstart(0, 0)
    gstart(1, 1)

    def pair(p, _):
        c0 = 2 * p
        gwait(0)
        sstart(c0, 0)
        gwait(1)
        swait(0)
        gstart(jnp.minimum(c0 + 2, NCH_S - 1), 0)
        sstart(c0 + 1, 1)
        swait(1)
        gstart(jnp.minimum(c0 + 3, NCH_S - 1), 1)
        return 0

    lax.fori_loop(0, NCH_S // 2, pair, 0)
    gwait(0)
    sstart(NCH_S - 1, 0)
    gwait(1)
    swait(0)
    plsc.subcore_barrier()
    pltpu.sync_copy(acc.at[pl.ds(sid * RPT, RPT)],
                    out_hbm.at[cid, pl.ds(sid * RPT, RPT)])


_R = 1024
_RF = 1000


def _dis3(d_refs):
    d0_ref, d1_ref = d_refs
    deg = d0_ref[0, :, 0:1] + d1_ref[0, :, 0:1] + 1.0
    return lax.rsqrt(deg)


def _mm_scale_body(d0_ref, d1_ref, x_ref, w_ref, o_ref):
    dis = _dis3((d0_ref, d1_ref))
    o_ref[...] = (dis * jnp.dot(x_ref[...], w_ref[0],
                                preferred_element_type=jnp.float32))[None]


def _mid_body(d0_ref, d1_ref, sa_ref, sb_ref, b_ref, w_ref, o_ref):
    dis = _dis3((d0_ref, d1_ref))
    za = dis * sa_ref[0] + b_ref[:, :DH]
    zb = dis * sb_ref[0] + b_ref[:, DH:]
    a = jnp.concatenate([jnp.maximum(za, 0.0), jnp.maximum(zb, 0.0)], axis=1)
    o_ref[...] = (dis * jnp.dot(a, w_ref[0],
                                preferred_element_type=jnp.float32))[None]


def _final_body(d0_ref, d1_ref, ta_ref, tb_ref, b_ref, o_ref):
    dis = _dis3((d0_ref, d1_ref))
    o_ref[...] = dis * jnp.concatenate(
        [ta_ref[0], tb_ref[0]], axis=1
    ) + b_ref[...]


def _dspec(r):
    return [pl.BlockSpec((1, r, 16), lambda i, *_: (0, i, 0)),
            pl.BlockSpec((1, r, 16), lambda i, *_: (1, i, 0))]


def _hspecs(r):
    return [pl.BlockSpec((1, r, DH), lambda i, *_: (0, i, 0)),
            pl.BlockSpec((1, r, DH), lambda i, *_: (1, i, 0))]


_H_OUT = jax.ShapeDtypeStruct((NC, NP, DH), jnp.float32)

_mm_scale = pl.pallas_call(
    _mm_scale_body,
    grid=(NP // _R, NC),
    in_specs=_dspec(_R) + [
        pl.BlockSpec((_R, D), lambda i, c: (i, 0)),
        pl.BlockSpec((1, D, DH), lambda i, c: (c, 0, 0)),
    ],
    out_specs=pl.BlockSpec((1, _R, DH), lambda i, c: (c, i, 0)),
    out_shape=_H_OUT,
)

_mid = pl.pallas_call(
    _mid_body,
    grid=(NP // _R, NC),
    in_specs=_dspec(_R) + _hspecs(_R) + [
        pl.BlockSpec((1, D), lambda i, c: (0, 0)),
        pl.BlockSpec((1, D, DH), lambda i, c: (c, 0, 0)),
    ],
    out_specs=pl.BlockSpec((1, _R, DH), lambda i, c: (c, i, 0)),
    out_shape=_H_OUT,
)

_final = pl.pallas_call(
    _final_body,
    grid=(N // _RF,),
    in_specs=_dspec(_RF) + _hspecs(_RF) + [
        pl.BlockSpec((1, D), lambda i: (0, 0)),
    ],
    out_specs=pl.BlockSpec((_RF, D), lambda i: (i, 0)),
    out_shape=jax.ShapeDtypeStruct((N, D), jnp.float32),
)


def kernel(x, edge_index, W1, b1, W2, b2):
    src_d = edge_index[0].astype(jnp.int32).reshape(NC * NS, NCH_D, K)
    dst_d = edge_index[1].astype(jnp.int32).reshape(NC * NS, NCH_D, K)
    src_s = edge_index[0].astype(jnp.int32).reshape(NS, EPT_S)
    dst_s = edge_index[1].astype(jnp.int32).reshape(NS, EPT_S)
    xp = jnp.pad(x, ((0, NP - N), (0, 0)))
    b1r = b1.reshape(1, D)
    b2r = b2.reshape(1, D)
    W1s = jnp.stack([W1[:, :DH], W1[:, DH:]])
    W2s = jnp.stack([W2[:, :DH], W2[:, DH:]])

    degp = _deg_kernel(dst_d)

    h1 = _mm_scale(degp, degp, xp, W1s)
    s = _scatter_kernel(h1, src_s, dst_s)
    h2 = _mid(degp, degp, s, s, b1r, W2s)
    t = _scatter_kernel(h2, src_s, dst_s)
    return _final(degp, degp, t, t, b2r)


# speedup vs baseline: 27.3872x; 1.0495x over previous
import functools

import jax
import jax.numpy as jnp
from jax import lax
from jax.experimental import pallas as pl
from jax.experimental.pallas import tpu as pltpu
from jax.experimental.pallas import tpu_sc as plsc

N = 10000
NP = 10240
D = 128
DH = D // 2
E = 320000
NC = 2
NS = 16
EPT_D = E // (NC * NS)
EPT_S = E // NS
K = 80
NCH_D = EPT_D // K
KB = 160
NCH_S = EPT_S // KB
RPT = NP // NS

_mesh = plsc.VectorSubcoreMesh(core_axis_name="c", subcore_axis_name="s")


@functools.partial(
    pl.kernel,
    out_type=jax.ShapeDtypeStruct((NC, NP, 16), jnp.float32),
    mesh=_mesh,
    compiler_params=pltpu.CompilerParams(use_tc_tiling_on_sc=False),
    scratch_types=[
        pltpu.VMEM((NCH_D, K), jnp.int32),
        pltpu.VMEM((K, 16), jnp.float32),
        pltpu.VMEM((128, 16), jnp.float32),
        pltpu.VMEM_SHARED((NP, 16), jnp.float32),
    ],
)
def _deg_kernel(dst_hbm, out_hbm, dstb, onesb, zbuf, acc):
    cid = lax.axis_index("c")
    sid = lax.axis_index("s")
    wid = sid * NC + cid

    one16 = jnp.ones((16,), jnp.float32)
    zero16 = jnp.zeros((16,), jnp.float32)

    def fill_ones(i, _):
        onesb[i] = one16
        return 0

    lax.fori_loop(0, K, fill_ones, 0)

    def fill_zero(i, _):
        zbuf[i] = zero16
        return 0

    lax.fori_loop(0, 128, fill_zero, 0)
    for b in range(RPT // 128):
        pltpu.sync_copy(zbuf, acc.at[pl.ds(sid * RPT + b * 128, 128)])
    plsc.subcore_barrier()

    pltpu.sync_copy(dst_hbm.at[wid], dstb)

    def body(j, _):
        pltpu.sync_copy(onesb, acc.at[dstb.at[j]], add=True)
        return 0

    lax.fori_loop(0, NCH_D, body, 0)
    plsc.subcore_barrier()
    pltpu.sync_copy(acc.at[pl.ds(sid * RPT, RPT)],
                    out_hbm.at[cid, pl.ds(sid * RPT, RPT)])


@functools.partial(
    pl.kernel,
    out_type=jax.ShapeDtypeStruct((NC, NP, DH), jnp.float32),
    mesh=_mesh,
    compiler_params=pltpu.CompilerParams(use_tc_tiling_on_sc=False),
    scratch_types=[
        pltpu.VMEM((EPT_S,), jnp.int32),
        pltpu.VMEM((EPT_S,), jnp.int32),
        pltpu.VMEM((4, KB, DH), jnp.float32),
        pltpu.VMEM_SHARED((NP, DH), jnp.float32),
        pltpu.SemaphoreType.DMA,
        pltpu.SemaphoreType.DMA,
    ],
)
def _scatter_kernel(h_hbm, src_hbm, dst_hbm, out_hbm, srcb, dstb, rows,
                    acc, gsem, ssem):
    cid = lax.axis_index("c")
    sid = lax.axis_index("s")

    pltpu.sync_copy(h_hbm.at[cid, pl.ds(sid * RPT, RPT)],
                    acc.at[pl.ds(sid * RPT, RPT)])
    plsc.subcore_barrier()

    pltpu.sync_copy(src_hbm.at[sid], srcb)
    pltpu.sync_copy(dst_hbm.at[sid], dstb)

    def gstart(j, buf):
        pltpu.async_copy(h_hbm.at[cid].at[srcb.at[pl.ds(j * KB, KB)]],
                         rows.at[buf], gsem)

    def gwait(buf):
        pltpu.make_async_copy(h_hbm.at[cid].at[srcb.at[pl.ds(0, KB)]],
                              rows.at[buf], gsem).wait()

    def sstart(j, buf):
        pltpu.async_copy(rows.at[buf], acc.at[dstb.at[pl.ds(j * KB, KB)]],
                         ssem, add=True)

    def swait(buf):
        pltpu.make_async_copy(rows.at[buf], acc.at[dstb.at[pl.ds(0, KB)]],
                              ssem).wait()

    for b in range(4):
        gstart(b, b)

    def quad(q, _):
        c0 = 4 * q
        for b in range(4):
            gwait(b)
            sstart(c0 + b, b)
        for b in range(4):
            swait(b)
            gstart(jnp.minimum(c0 + 4 + b, NCH_S - 1), b)
        return 0

    lax.fori_loop(0, NCH_S // 4, quad, 0)
    gwait(0)
    sstart(NCH_S - 1, 0)
    gwait(1)
    gwait(2)
    gwait(3)
    swait(0)
    plsc.subcore_barrier()
    pltpu.sync_copy(acc.at[pl.ds(sid * RPT, RPT)],
                    out_hbm.at[cid, pl.ds(sid * RPT, RPT)])


_R = 1024
_RF = 1000


def _dis3(d_refs):
    d0_ref, d1_ref = d_refs
    deg = d0_ref[0, :, 0:1] + d1_ref[0, :, 0:1] + 1.0
    return lax.rsqrt(deg)


def _mm_scale_body(d0_ref, d1_ref, x_ref, w_ref, o_ref):
    dis = _dis3((d0_ref, d1_ref))
    o_ref[...] = (dis * jnp.dot(x_ref[...], w_ref[0],
                                preferred_element_type=jnp.float32))[None]


def _mid_body(d0_ref, d1_ref, sa_ref, sb_ref, b_ref, w_ref, o_ref):
    dis = _dis3((d0_ref, d1_ref))
    za = dis * sa_ref[0] + b_ref[:, :DH]
    zb = dis * sb_ref[0] + b_ref[:, DH:]
    a = jnp.concatenate([jnp.maximum(za, 0.0), jnp.maximum(zb, 0.0)], axis=1)
    o_ref[...] = (dis * jnp.dot(a, w_ref[0],
                                preferred_element_type=jnp.float32))[None]


def _final_body(d0_ref, d1_ref, ta_ref, tb_ref, b_ref, o_ref):
    dis = _dis3((d0_ref, d1_ref))
    o_ref[...] = dis * jnp.concatenate(
        [ta_ref[0], tb_ref[0]], axis=1
    ) + b_ref[...]


def _dspec(r):
    return [pl.BlockSpec((1, r, 16), lambda i, *_: (0, i, 0)),
            pl.BlockSpec((1, r, 16), lambda i, *_: (1, i, 0))]


def _hspecs(r):
    return [pl.BlockSpec((1, r, DH), lambda i, *_: (0, i, 0)),
            pl.BlockSpec((1, r, DH), lambda i, *_: (1, i, 0))]


_H_OUT = jax.ShapeDtypeStruct((NC, NP, DH), jnp.float32)

_mm_scale = pl.pallas_call(
    _mm_scale_body,
    grid=(NP // _R, NC),
    in_specs=_dspec(_R) + [
        pl.BlockSpec((_R, D), lambda i, c: (i, 0)),
        pl.BlockSpec((1, D, DH), lambda i, c: (c, 0, 0)),
    ],
    out_specs=pl.BlockSpec((1, _R, DH), lambda i, c: (c, i, 0)),
    out_shape=_H_OUT,
)

_mid = pl.pallas_call(
    _mid_body,
    grid=(NP // _R, NC),
    in_specs=_dspec(_R) + _hspecs(_R) + [
        pl.BlockSpec((1, D), lambda i, c: (0, 0)),
        pl.BlockSpec((1, D, DH), lambda i, c: (c, 0, 0)),
    ],
    out_specs=pl.BlockSpec((1, _R, DH), lambda i, c: (c, i, 0)),
    out_shape=_H_OUT,
)

_final = pl.pallas_call(
    _final_body,
    grid=(N // _RF,),
    in_specs=_dspec(_RF) + _hspecs(_RF) + [
        pl.BlockSpec((1, D), lambda i: (0, 0)),
    ],
    out_specs=pl.BlockSpec((_RF, D), lambda i: (i, 0)),
    out_shape=jax.ShapeDtypeStruct((N, D), jnp.float32),
)


def kernel(x, edge_index, W1, b1, W2, b2):
    src_d = edge_index[0].astype(jnp.int32).reshape(NC * NS, NCH_D, K)
    dst_d = edge_index[1].astype(jnp.int32).reshape(NC * NS, NCH_D, K)
    src_s = edge_index[0].astype(jnp.int32).reshape(NS, EPT_S)
    dst_s = edge_index[1].astype(jnp.int32).reshape(NS, EPT_S)
    xp = jnp.pad(x, ((0, NP - N), (0, 0)))
    b1r = b1.reshape(1, D)
    b2r = b2.reshape(1, D)
    W1s = jnp.stack([W1[:, :DH], W1[:, DH:]])
    W2s = jnp.stack([W2[:, :DH], W2[:, DH:]])

    degp = _deg_kernel(dst_d)

    h1 = _mm_scale(degp, degp, xp, W1s)
    s = _scatter_kernel(h1, src_s, dst_s)
    h2 = _mid(degp, degp, s, s, b1r, W2s)
    t = _scatter_kernel(h2, src_s, dst_s)
    return _final(degp, degp, t, t, b2r)


# speedup vs baseline: 27.4040x; 1.0006x over previous
import functools

import jax
import jax.numpy as jnp
from jax import lax
from jax.experimental import pallas as pl
from jax.experimental.pallas import tpu as pltpu
from jax.experimental.pallas import tpu_sc as plsc

N = 10000
NP = 10240
D = 128
DH = D // 2
E = 320000
NC = 2
NS = 16
EPT_D = E // (NC * NS)
EPT_S = E // NS
K = 80
NCH_D = EPT_D // K
KB = 200
NCH_S = EPT_S // KB
RPT = NP // NS

_mesh = plsc.VectorSubcoreMesh(core_axis_name="c", subcore_axis_name="s")


@functools.partial(
    pl.kernel,
    out_type=jax.ShapeDtypeStruct((NC, NP, 16), jnp.float32),
    mesh=_mesh,
    compiler_params=pltpu.CompilerParams(use_tc_tiling_on_sc=False),
    scratch_types=[
        pltpu.VMEM((NCH_D, K), jnp.int32),
        pltpu.VMEM((K, 16), jnp.float32),
        pltpu.VMEM((128, 16), jnp.float32),
        pltpu.VMEM_SHARED((NP, 16), jnp.float32),
    ],
)
def _deg_kernel(dst_hbm, out_hbm, dstb, onesb, zbuf, acc):
    cid = lax.axis_index("c")
    sid = lax.axis_index("s")
    wid = sid * NC + cid

    one16 = jnp.ones((16,), jnp.float32)
    zero16 = jnp.zeros((16,), jnp.float32)

    def fill_ones(i, _):
        onesb[i] = one16
        return 0

    lax.fori_loop(0, K, fill_ones, 0)

    def fill_zero(i, _):
        zbuf[i] = zero16
        return 0

    lax.fori_loop(0, 128, fill_zero, 0)
    for b in range(RPT // 128):
        pltpu.sync_copy(zbuf, acc.at[pl.ds(sid * RPT + b * 128, 128)])
    plsc.subcore_barrier()

    pltpu.sync_copy(dst_hbm.at[wid], dstb)

    def body(j, _):
        pltpu.sync_copy(onesb, acc.at[dstb.at[j]], add=True)
        return 0

    lax.fori_loop(0, NCH_D, body, 0)
    plsc.subcore_barrier()
    pltpu.sync_copy(acc.at[pl.ds(sid * RPT, RPT)],
                    out_hbm.at[cid, pl.ds(sid * RPT, RPT)])


@functools.partial(
    pl.kernel,
    out_type=jax.ShapeDtypeStruct((NC, NP, DH), jnp.float32),
    mesh=_mesh,
    compiler_params=pltpu.CompilerParams(use_tc_tiling_on_sc=False),
    scratch_types=[
        pltpu.VMEM((EPT_S,), jnp.int32),
        pltpu.VMEM((EPT_S,), jnp.int32),
        pltpu.VMEM((3, KB, DH), jnp.float32),
        pltpu.VMEM_SHARED((NP, DH), jnp.float32),
        pltpu.SemaphoreType.DMA,
        pltpu.SemaphoreType.DMA,
    ],
)
def _scatter_kernel(h_hbm, src_hbm, dst_hbm, out_hbm, srcb, dstb, rows,
                    acc, gsem, ssem):
    cid = lax.axis_index("c")
    sid = lax.axis_index("s")

    pltpu.sync_copy(h_hbm.at[cid, pl.ds(sid * RPT, RPT)],
                    acc.at[pl.ds(sid * RPT, RPT)])
    plsc.subcore_barrier()

    pltpu.sync_copy(src_hbm.at[sid], srcb)
    pltpu.sync_copy(dst_hbm.at[sid], dstb)

    def gstart(j, buf):
        pltpu.async_copy(h_hbm.at[cid].at[srcb.at[pl.ds(j * KB, KB)]],
                         rows.at[buf], gsem)

    def gwait(buf):
        pltpu.make_async_copy(h_hbm.at[cid].at[srcb.at[pl.ds(0, KB)]],
                              rows.at[buf], gsem).wait()

    def sstart(j, buf):
        pltpu.async_copy(rows.at[buf], acc.at[dstb.at[pl.ds(j * KB, KB)]],
                         ssem, add=True)

    def swait(buf):
        pltpu.make_async_copy(rows.at[buf], acc.at[dstb.at[pl.ds(0, KB)]],
                              ssem).wait()

    for b in range(3):
        gstart(b, b)

    def triple(q, _):
        c0 = 3 * q
        for b in range(3):
            gwait(b)
            sstart(c0 + b, b)
        for b in range(3):
            swait(b)
            gstart(jnp.minimum(c0 + 3 + b, NCH_S - 1), b)
        return 0

    lax.fori_loop(0, NCH_S // 3, triple, 0)
    gwait(0)
    sstart(NCH_S - 1, 0)
    gwait(1)
    gwait(2)
    swait(0)
    plsc.subcore_barrier()
    pltpu.sync_copy(acc.at[pl.ds(sid * RPT, RPT)],
                    out_hbm.at[cid, pl.ds(sid * RPT, RPT)])


_R = 1024
_RF = 1000


def _dis3(d_refs):
    d0_ref, d1_ref = d_refs
    deg = d0_ref[0, :, 0:1] + d1_ref[0, :, 0:1] + 1.0
    return lax.rsqrt(deg)


def _mm_scale_body(d0_ref, d1_ref, x_ref, w_ref, o_ref):
    dis = _dis3((d0_ref, d1_ref))
    o_ref[...] = (dis * jnp.dot(x_ref[...], w_ref[0],
                                preferred_element_type=jnp.float32))[None]


def _mid_body(d0_ref, d1_ref, sa_ref, sb_ref, b_ref, w_ref, o_ref):
    dis = _dis3((d0_ref, d1_ref))
    za = dis * sa_ref[0] + b_ref[:, :DH]
    zb = dis * sb_ref[0] + b_ref[:, DH:]
    a = jnp.concatenate([jnp.maximum(za, 0.0), jnp.maximum(zb, 0.0)], axis=1)
    o_ref[...] = (dis * jnp.dot(a, w_ref[0],
                                preferred_element_type=jnp.float32))[None]


def _final_body(d0_ref, d1_ref, ta_ref, tb_ref, b_ref, o_ref):
    dis = _dis3((d0_ref, d1_ref))
    o_ref[...] = dis * jnp.concatenate(
        [ta_ref[0], tb_ref[0]], axis=1
    ) + b_ref[...]


def _dspec(r):
    return [pl.BlockSpec((1, r, 16), lambda i, *_: (0, i, 0)),
            pl.BlockSpec((1, r, 16), lambda i, *_: (1, i, 0))]


def _hspecs(r):
    return [pl.BlockSpec((1, r, DH), lambda i, *_: (0, i, 0)),
            pl.BlockSpec((1, r, DH), lambda i, *_: (1, i, 0))]


_H_OUT = jax.ShapeDtypeStruct((NC, NP, DH), jnp.float32)

_mm_scale = pl.pallas_call(
    _mm_scale_body,
    grid=(NP // _R, NC),
    in_specs=_dspec(_R) + [
        pl.BlockSpec((_R, D), lambda i, c: (i, 0)),
        pl.BlockSpec((1, D, DH), lambda i, c: (c, 0, 0)),
    ],
    out_specs=pl.BlockSpec((1, _R, DH), lambda i, c: (c, i, 0)),
    out_shape=_H_OUT,
)

_mid = pl.pallas_call(
    _mid_body,
    grid=(NP // _R, NC),
    in_specs=_dspec(_R) + _hspecs(_R) + [
        pl.BlockSpec((1, D), lambda i, c: (0, 0)),
        pl.BlockSpec((1, D, DH), lambda i, c: (c, 0, 0)),
    ],
    out_specs=pl.BlockSpec((1, _R, DH), lambda i, c: (c, i, 0)),
    out_shape=_H_OUT,
)

_final = pl.pallas_call(
    _final_body,
    grid=(N // _RF,),
    in_specs=_dspec(_RF) + _hspecs(_RF) + [
        pl.BlockSpec((1, D), lambda i: (0, 0)),
    ],
    out_specs=pl.BlockSpec((_RF, D), lambda i: (i, 0)),
    out_shape=jax.ShapeDtypeStruct((N, D), jnp.float32),
)


def kernel(x, edge_index, W1, b1, W2, b2):
    src_d = edge_index[0].astype(jnp.int32).reshape(NC * NS, NCH_D, K)
    dst_d = edge_index[1].astype(jnp.int32).reshape(NC * NS, NCH_D, K)
    src_s = edge_index[0].astype(jnp.int32).reshape(NS, EPT_S)
    dst_s = edge_index[1].astype(jnp.int32).reshape(NS, EPT_S)
    xp = jnp.pad(x, ((0, NP - N), (0, 0)))
    b1r = b1.reshape(1, D)
    b2r = b2.reshape(1, D)
    W1s = jnp.stack([W1[:, :DH], W1[:, DH:]])
    W2s = jnp.stack([W2[:, :DH], W2[:, DH:]])

    degp = _deg_kernel(dst_d)

    h1 = _mm_scale(degp, degp, xp, W1s)
    s = _scatter_kernel(h1, src_s, dst_s)
    h2 = _mid(degp, degp, s, s, b1r, W2s)
    t = _scatter_kernel(h2, src_s, dst_s)
    return _final(degp, degp, t, t, b2r)
